# baseline (speedup 1.0000x reference)
import functools

import jax
import jax.numpy as jnp
from jax import lax
from jax.experimental import pallas as pl
from jax.experimental.pallas import tpu as pltpu
from jax.experimental.pallas import tpu_sc as plsc

_N = 10000
_D = 128
_E = 320000
_C = 64
_EPAD = 327680
_ER = _EPAD // _C
_RPT = _ER // 32
_LSTG = 16
_NPAD = 10240

_mesh = plsc.VectorSubcoreMesh(core_axis_name="c", subcore_axis_name="s")

_Z16 = functools.partial(jnp.zeros, (16,), jnp.float32)


@functools.partial(
    pl.kernel,
    out_type=jax.ShapeDtypeStruct((2 * _NPAD,), jnp.float32),
    mesh=_mesh,
    scratch_types=[
        pltpu.VMEM((_RPT, _C), jnp.int32),
        pltpu.VMEM((_RPT, _C), jnp.float32),
        pltpu.VMEM((640,), jnp.float32),
        pltpu.VMEM_SHARED((_NPAD,), jnp.float32),
    ],
)
def _deg(src_hbm, w_hbm, out_hbm, sidx, w_all, zbuf, acc):
    cid = lax.axis_index("c")
    sid = lax.axis_index("s")
    wid = cid * 16 + sid
    for j in range(40):
        zbuf[pl.ds(j * 16, 16)] = _Z16()
    off = pl.multiple_of(sid * 640, 8)
    pltpu.sync_copy(zbuf, acc.at[pl.ds(off, 640)])
    plsc.subcore_barrier()
    rbase = pl.multiple_of(wid * _RPT, 1)
    pltpu.sync_copy(src_hbm.at[pl.ds(rbase, _RPT)], sidx)
    pltpu.sync_copy(w_hbm.at[pl.ds(rbase, _RPT)], w_all)

    def chunk(k, carry):
        pltpu.sync_copy(w_all.at[k], acc.at[sidx.at[k]], add=True)
        return carry

    lax.fori_loop(0, _RPT, chunk, 0)
    plsc.subcore_barrier()
    obase = pl.multiple_of(cid * _NPAD + sid * 640, 8)
    pltpu.sync_copy(acc.at[pl.ds(off, 640)], out_hbm.at[pl.ds(obase, 640)])


@functools.partial(
    pl.kernel,
    out_type=jax.ShapeDtypeStruct((_ER, _C), jnp.float32),
    mesh=_mesh,
    scratch_types=[
        pltpu.VMEM((_RPT, _C), jnp.int32),
        pltpu.VMEM((_RPT, _C), jnp.int32),
        pltpu.VMEM((_RPT, _C), jnp.float32),
        pltpu.VMEM((_RPT, _C), jnp.float32),
        pltpu.VMEM((_C,), jnp.float32),
        pltpu.VMEM((_C,), jnp.float32),
        pltpu.VMEM_SHARED((_NPAD,), jnp.float32),
    ],
)
def _norm(dis_hbm, src_hbm, dst_hbm, w_hbm, out_hbm,
          sidx, didx, w_all, nnbuf, sv, dg, sdis):
    cid = lax.axis_index("c")
    sid = lax.axis_index("s")
    wid = cid * 16 + sid
    off = pl.multiple_of(sid * 640, 8)
    pltpu.sync_copy(dis_hbm.at[pl.ds(off, 640)], sdis.at[pl.ds(off, 640)])
    plsc.subcore_barrier()
    rbase = pl.multiple_of(wid * _RPT, 1)
    pltpu.sync_copy(src_hbm.at[pl.ds(rbase, _RPT)], sidx)
    pltpu.sync_copy(dst_hbm.at[pl.ds(rbase, _RPT)], didx)
    pltpu.sync_copy(w_hbm.at[pl.ds(rbase, _RPT)], w_all)

    def chunk(k, carry):
        pltpu.sync_copy(sdis.at[sidx.at[k]], sv)
        pltpu.sync_copy(sdis.at[didx.at[k]], dg)
        for j in range(_C // 16):
            s = sv[pl.ds(j * 16, 16)]
            dd = dg[pl.ds(j * 16, 16)]
            ww = w_all[k, pl.ds(j * 16, 16)]
            nnbuf[k, pl.ds(j * 16, 16)] = -(s * ww * dd)
        return carry

    lax.fori_loop(0, _RPT, chunk, 0)
    pltpu.sync_copy(nnbuf, out_hbm.at[pl.ds(rbase, _RPT)])


@functools.partial(
    pl.kernel,
    out_type=jax.ShapeDtypeStruct((2 * _NPAD, _D), jnp.float32),
    mesh=_mesh,
    scratch_types=[
        pltpu.VMEM((_LSTG, _C), jnp.int32),
        pltpu.VMEM((_LSTG, _C), jnp.int32),
        pltpu.VMEM((_LSTG, _C), jnp.float32),
        pltpu.VMEM((_C, _D), jnp.float32),
        pltpu.VMEM((_C, _D), jnp.float32),
        pltpu.SemaphoreType.DMA,
        pltpu.SemaphoreType.DMA,
        pltpu.SemaphoreType.DMA,
        pltpu.VMEM_SHARED((_NPAD, _D), jnp.float32),
    ],
)
def _lap(v_hbm, src_hbm, dst_hbm, nn_hbm, out_hbm,
         sidx, didx, nn_all, rows0, rows1, sem0, sem1, sems0, acc):
    cid = lax.axis_index("c")
    sid = lax.axis_index("s")
    wid = cid * 16 + sid

    def zrow(i, carry):
        for j in range(_D // 16):
            rows0[i, pl.ds(j * 16, 16)] = _Z16()
        return carry

    lax.fori_loop(0, _C, zrow, 0)
    for t in range(640 // _C):
        pltpu.sync_copy(
            rows0, acc.at[pl.ds(pl.multiple_of(sid * 640 + t * _C, 8), _C)])
    plsc.subcore_barrier()
    rbase = pl.multiple_of(wid * _RPT, 8)

    def scale(rows, ks):
        for g in range(_C // 16):
            nv = nn_all[ks, pl.ds(g * 16, 16)]
            for lane in range(16):
                ri = g * 16 + lane
                sc = nv[lane]
                for j in range(_D // 16):
                    rows[ri, pl.ds(j * 16, 16)] = rows[ri, pl.ds(j * 16, 16)] * sc

    def pair(kk, carry):
        ks0 = (2 * kk) % _LSTG
        ks1 = ks0 + 1

        @pl.when(ks0 == 0)
        def _stage():
            rb = pl.multiple_of(rbase + 2 * kk, 8)
            pltpu.sync_copy(src_hbm.at[pl.ds(rb, _LSTG)], sidx)
            pltpu.sync_copy(dst_hbm.at[pl.ds(rb, _LSTG)], didx)
            pltpu.sync_copy(nn_hbm.at[pl.ds(rb, _LSTG)], nn_all)
            pltpu.async_copy(v_hbm.at[sidx.at[ks0]], rows0, sem0)
            pltpu.async_copy(v_hbm.at[sidx.at[ks1]], rows1, sem1)

        pltpu.make_async_copy(v_hbm.at[sidx.at[ks0]], rows0, sem0).wait()
        scale(rows0, ks0)
        pltpu.async_copy(rows0, acc.at[didx.at[ks0]], sems0, add=True)
        pltpu.make_async_copy(v_hbm.at[sidx.at[ks1]], rows1, sem1).wait()
        scale(rows1, ks1)
        pltpu.make_async_copy(rows0, acc.at[didx.at[ks0]], sems0).wait()

        @pl.when(ks0 + 2 < _LSTG)
        def _pref0():
            pltpu.async_copy(v_hbm.at[sidx.at[ks0 + 2]], rows0, sem0)

        pltpu.sync_copy(rows1, acc.at[didx.at[ks1]], add=True)

        @pl.when(ks0 + 3 < _LSTG)
        def _pref1():
            pltpu.async_copy(v_hbm.at[sidx.at[ks0 + 3]], rows1, sem1)

        return carry

    lax.fori_loop(0, _RPT // 2, pair, 0)
    plsc.subcore_barrier()
    pltpu.sync_copy(acc.at[pl.ds(pl.multiple_of(sid * 640, 8), 640)],
                    out_hbm.at[pl.ds(pl.multiple_of(cid * _NPAD + sid * 640, 8), 640)])


def _dis_body(dp, o):
    d = dp[0:8, :] + dp[8:16, :]
    o[...] = jnp.where(d > 0.0, lax.rsqrt(jnp.where(d > 0.0, d, 1.0)), 0.0)


_dis = pl.pallas_call(
    _dis_body,
    grid=(1,),
    in_specs=[pl.BlockSpec((16, 1280), lambda i: (0, 0))],
    out_specs=pl.BlockSpec((8, 1280), lambda i: (0, 0)),
    out_shape=jax.ShapeDtypeStruct((8, 1280), jnp.float32),
)


_BN = 512
_NB = 20
_PB = _NPAD // _BN


def _comb_body(a, b, o):
    o[...] = a[...] + b[...]


_comb = pl.pallas_call(
    _comb_body,
    grid=(_NB,),
    in_specs=[
        pl.BlockSpec((_BN, _D), lambda i: (i, 0)),
        pl.BlockSpec((_BN, _D), lambda i: (i + _PB, 0)),
    ],
    out_specs=pl.BlockSpec((_BN, _D), lambda i: (i, 0)),
    out_shape=jax.ShapeDtypeStruct((_N, _D), jnp.float32),
)


def _six_matmul(terms, w, bias):
    acc = bias
    for i in range(6):
        acc = acc + jnp.dot(terms[i], w[i], preferred_element_type=jnp.float32)
    return acc


def _dense1_body(x, u1, u2a, u2b, h, v1, v2a, v2b, wz, wr, bz, br, z_o, hr_o):
    xx = x[...]
    hh = h[...]
    terms = [xx, u1[...], u2a[...] + u2b[...], hh, v1[...], v2a[...] + v2b[...]]
    z = jax.nn.sigmoid(_six_matmul(terms, wz[...], bz[...]))
    r = jax.nn.sigmoid(_six_matmul(terms, wr[...], br[...]))
    z_o[...] = z
    hr_o[...] = hh * r


_dense1 = pl.pallas_call(
    _dense1_body,
    grid=(_NB,),
    in_specs=[
        pl.BlockSpec((_BN, _D), lambda i: (i, 0)),
        pl.BlockSpec((_BN, _D), lambda i: (i, 0)),
        pl.BlockSpec((_BN, _D), lambda i: (i, 0)),
        pl.BlockSpec((_BN, _D), lambda i: (i + _PB, 0)),
        pl.BlockSpec((_BN, _D), lambda i: (i, 0)),
        pl.BlockSpec((_BN, _D), lambda i: (i, 0)),
        pl.BlockSpec((_BN, _D), lambda i: (i, 0)),
        pl.BlockSpec((_BN, _D), lambda i: (i + _PB, 0)),
        pl.BlockSpec((6, _D, _D), lambda i: (0, 0, 0)),
        pl.BlockSpec((6, _D, _D), lambda i: (0, 0, 0)),
        pl.BlockSpec((1, _D), lambda i: (0, 0)),
        pl.BlockSpec((1, _D), lambda i: (0, 0)),
    ],
    out_specs=[
        pl.BlockSpec((_BN, _D), lambda i: (i, 0)),
        pl.BlockSpec((_BN, _D), lambda i: (i, 0)),
    ],
    out_shape=[
        jax.ShapeDtypeStruct((_N, _D), jnp.float32),
        jax.ShapeDtypeStruct((_N, _D), jnp.float32),
    ],
)


def _dense2_body(x, u1, u2a, u2b, hr, p1, p2a, p2b, z, h, wh, bh, o):
    terms = [x[...], u1[...], u2a[...] + u2b[...],
             hr[...], p1[...], p2a[...] + p2b[...]]
    ht = jnp.tanh(_six_matmul(terms, wh[...], bh[...]))
    zz = z[...]
    o[...] = zz * h[...] + (1.0 - zz) * ht


_dense2 = pl.pallas_call(
    _dense2_body,
    grid=(_NB,),
    in_specs=[
        pl.BlockSpec((_BN, _D), lambda i: (i, 0)),
        pl.BlockSpec((_BN, _D), lambda i: (i, 0)),
        pl.BlockSpec((_BN, _D), lambda i: (i, 0)),
        pl.BlockSpec((_BN, _D), lambda i: (i + _PB, 0)),
        pl.BlockSpec((_BN, _D), lambda i: (i, 0)),
        pl.BlockSpec((_BN, _D), lambda i: (i, 0)),
        pl.BlockSpec((_BN, _D), lambda i: (i, 0)),
        pl.BlockSpec((_BN, _D), lambda i: (i + _PB, 0)),
        pl.BlockSpec((_BN, _D), lambda i: (i, 0)),
        pl.BlockSpec((_BN, _D), lambda i: (i, 0)),
        pl.BlockSpec((6, _D, _D), lambda i: (0, 0, 0)),
        pl.BlockSpec((1, _D), lambda i: (0, 0)),
    ],
    out_specs=pl.BlockSpec((_BN, _D), lambda i: (i, 0)),
    out_shape=jax.ShapeDtypeStruct((_N, _D), jnp.float32),
)


def _gate_weights(Wx, Wh):
    return jnp.stack([Wx[0] - Wx[2], Wx[1], 2.0 * Wx[2],
                      Wh[0] - Wh[2], Wh[1], 2.0 * Wh[2]])


def kernel(X, edge_index, edge_weight, H,
           Wxz, bxz, Whz, bhz, Wxr, bxr, Whr, bhr, Wxh, bxh, Whh, bhh):
    pad = _EPAD - _E
    zi = jnp.zeros((pad,), jnp.int32)
    src2 = jnp.concatenate([edge_index[0], zi]).reshape(_ER, _C)
    dst2 = jnp.concatenate([edge_index[1], zi]).reshape(_ER, _C)
    w2 = jnp.concatenate([edge_weight, jnp.zeros((pad,), jnp.float32)]).reshape(_ER, _C)

    degp = _deg(src2, w2)
    dis = _dis(degp.reshape(16, 1280)).reshape(_NPAD)
    nn2 = _norm(dis, src2, dst2, w2)

    u1p = _lap(X, src2, dst2, nn2)
    u1 = _comb(u1p, u1p)
    u2p = _lap(u1, src2, dst2, nn2)
    v1p = _lap(H, src2, dst2, nn2)
    v1 = _comb(v1p, v1p)
    v2p = _lap(v1, src2, dst2, nn2)

    Wz = _gate_weights(Wxz, Whz)
    Wr = _gate_weights(Wxr, Whr)
    Wh = _gate_weights(Wxh, Whh)
    bz = (bxz + bhz).reshape(1, _D)
    br = (bxr + bhr).reshape(1, _D)
    bh = (bxh + bhh).reshape(1, _D)

    Z, HR = _dense1(X, u1, u2p, u2p, H, v1, v2p, v2p, Wz, Wr, bz, br)

    p1p = _lap(HR, src2, dst2, nn2)
    p1 = _comb(p1p, p1p)
    p2p = _lap(p1, src2, dst2, nn2)

    return _dense2(X, u1, u2p, u2p, HR, p1, p2p, p2p, Z, H, Wh, bh)


# speedup vs baseline: 1.1273x; 1.1273x over previous
import functools

import jax
import jax.numpy as jnp
from jax import lax
from jax.experimental import pallas as pl
from jax.experimental.pallas import tpu as pltpu
from jax.experimental.pallas import tpu_sc as plsc

_N = 10000
_D = 128
_E = 320000
_C = 80
_EPAD = 327680
_ER = _EPAD // _C
_RPT = _ER // 32
_LSTG = 8
_NPAD = 10240

_mesh = plsc.VectorSubcoreMesh(core_axis_name="c", subcore_axis_name="s")

_Z16 = functools.partial(jnp.zeros, (16,), jnp.float32)


@functools.partial(
    pl.kernel,
    out_type=jax.ShapeDtypeStruct((2 * _NPAD,), jnp.float32),
    mesh=_mesh,
    scratch_types=[
        pltpu.VMEM((_RPT, _C), jnp.int32),
        pltpu.VMEM((_RPT, _C), jnp.float32),
        pltpu.VMEM((640,), jnp.float32),
        pltpu.VMEM_SHARED((_NPAD,), jnp.float32),
    ],
)
def _deg(src_hbm, w_hbm, out_hbm, sidx, w_all, zbuf, acc):
    cid = lax.axis_index("c")
    sid = lax.axis_index("s")
    wid = cid * 16 + sid
    for j in range(40):
        zbuf[pl.ds(j * 16, 16)] = _Z16()
    off = pl.multiple_of(sid * 640, 8)
    pltpu.sync_copy(zbuf, acc.at[pl.ds(off, 640)])
    plsc.subcore_barrier()
    rbase = pl.multiple_of(wid * _RPT, 1)
    pltpu.sync_copy(src_hbm.at[pl.ds(rbase, _RPT)], sidx)
    pltpu.sync_copy(w_hbm.at[pl.ds(rbase, _RPT)], w_all)

    def chunk(k, carry):
        pltpu.sync_copy(w_all.at[k], acc.at[sidx.at[k]], add=True)
        return carry

    lax.fori_loop(0, _RPT, chunk, 0)
    plsc.subcore_barrier()
    obase = pl.multiple_of(cid * _NPAD + sid * 640, 8)
    pltpu.sync_copy(acc.at[pl.ds(off, 640)], out_hbm.at[pl.ds(obase, 640)])


@functools.partial(
    pl.kernel,
    out_type=jax.ShapeDtypeStruct((_ER, _C), jnp.float32),
    mesh=_mesh,
    scratch_types=[
        pltpu.VMEM((_RPT, _C), jnp.int32),
        pltpu.VMEM((_RPT, _C), jnp.int32),
        pltpu.VMEM((_RPT, _C), jnp.float32),
        pltpu.VMEM((_RPT, _C), jnp.float32),
        pltpu.VMEM((_C,), jnp.float32),
        pltpu.VMEM((_C,), jnp.float32),
        pltpu.VMEM_SHARED((_NPAD,), jnp.float32),
    ],
)
def _norm(dis_hbm, src_hbm, dst_hbm, w_hbm, out_hbm,
          sidx, didx, w_all, nnbuf, sv, dg, sdis):
    cid = lax.axis_index("c")
    sid = lax.axis_index("s")
    wid = cid * 16 + sid
    off = pl.multiple_of(sid * 640, 8)
    pltpu.sync_copy(dis_hbm.at[pl.ds(off, 640)], sdis.at[pl.ds(off, 640)])
    plsc.subcore_barrier()
    rbase = pl.multiple_of(wid * _RPT, 1)
    pltpu.sync_copy(src_hbm.at[pl.ds(rbase, _RPT)], sidx)
    pltpu.sync_copy(dst_hbm.at[pl.ds(rbase, _RPT)], didx)
    pltpu.sync_copy(w_hbm.at[pl.ds(rbase, _RPT)], w_all)

    def chunk(k, carry):
        pltpu.sync_copy(sdis.at[sidx.at[k]], sv)
        pltpu.sync_copy(sdis.at[didx.at[k]], dg)
        for j in range(_C // 16):
            s = sv[pl.ds(j * 16, 16)]
            dd = dg[pl.ds(j * 16, 16)]
            ww = w_all[k, pl.ds(j * 16, 16)]
            nnbuf[k, pl.ds(j * 16, 16)] = -(s * ww * dd)
        return carry

    lax.fori_loop(0, _RPT, chunk, 0)
    pltpu.sync_copy(nnbuf, out_hbm.at[pl.ds(rbase, _RPT)])


@functools.partial(
    pl.kernel,
    out_type=jax.ShapeDtypeStruct((2 * _NPAD, _D), jnp.float32),
    mesh=_mesh,
    scratch_types=[
        pltpu.VMEM((_LSTG, _C), jnp.int32),
        pltpu.VMEM((_LSTG, _C), jnp.int32),
        pltpu.VMEM((_LSTG, _C), jnp.float32),
        pltpu.VMEM((_C, _D), jnp.float32),
        pltpu.VMEM((_C, _D), jnp.float32),
        pltpu.SemaphoreType.DMA,
        pltpu.SemaphoreType.DMA,
        pltpu.VMEM_SHARED((_NPAD, _D), jnp.float32),
    ],
)
def _lap(v_hbm, src_hbm, dst_hbm, nn_hbm, out_hbm,
         sidx, didx, nn_all, rows0, rows1, sem0, sem1, acc):
    cid = lax.axis_index("c")
    sid = lax.axis_index("s")
    wid = cid * 16 + sid

    def zrow(i, carry):
        for j in range(_D // 16):
            rows0[i, pl.ds(j * 16, 16)] = _Z16()
        return carry

    lax.fori_loop(0, _C, zrow, 0)
    for t in range(640 // _C):
        pltpu.sync_copy(
            rows0, acc.at[pl.ds(pl.multiple_of(sid * 640 + t * _C, 8), _C)])
    plsc.subcore_barrier()
    rbase = pl.multiple_of(wid * _RPT, 8)

    def scale(rows, ks):
        for g in range(_C // 16):
            nv = nn_all[ks, pl.ds(g * 16, 16)]
            for lane in range(16):
                ri = g * 16 + lane
                sc = nv[lane]
                for j in range(_D // 16):
                    rows[ri, pl.ds(j * 16, 16)] = rows[ri, pl.ds(j * 16, 16)] * sc

    def pair(kk, carry):
        ks0 = (2 * kk) % _LSTG
        ks1 = ks0 + 1

        @pl.when(ks0 == 0)
        def _stage():
            rb = pl.multiple_of(rbase + 2 * kk, 8)
            pltpu.sync_copy(src_hbm.at[pl.ds(rb, _LSTG)], sidx)
            pltpu.sync_copy(dst_hbm.at[pl.ds(rb, _LSTG)], didx)
            pltpu.sync_copy(nn_hbm.at[pl.ds(rb, _LSTG)], nn_all)
            pltpu.async_copy(v_hbm.at[sidx.at[ks0]], rows0, sem0)
            pltpu.async_copy(v_hbm.at[sidx.at[ks1]], rows1, sem1)

        pltpu.make_async_copy(v_hbm.at[sidx.at[ks0]], rows0, sem0).wait()
        scale(rows0, ks0)
        pltpu.sync_copy(rows0, acc.at[didx.at[ks0]], add=True)

        @pl.when(ks0 + 2 < _LSTG)
        def _pref0():
            pltpu.async_copy(v_hbm.at[sidx.at[ks0 + 2]], rows0, sem0)

        pltpu.make_async_copy(v_hbm.at[sidx.at[ks1]], rows1, sem1).wait()
        scale(rows1, ks1)
        pltpu.sync_copy(rows1, acc.at[didx.at[ks1]], add=True)

        @pl.when(ks0 + 3 < _LSTG)
        def _pref1():
            pltpu.async_copy(v_hbm.at[sidx.at[ks0 + 3]], rows1, sem1)

        return carry

    lax.fori_loop(0, _RPT // 2, pair, 0)
    plsc.subcore_barrier()
    pltpu.sync_copy(acc.at[pl.ds(pl.multiple_of(sid * 640, 8), 640)],
                    out_hbm.at[pl.ds(pl.multiple_of(cid * _NPAD + sid * 640, 8), 640)])


def _dis_body(dp, o):
    d = dp[0:8, :] + dp[8:16, :]
    o[...] = jnp.where(d > 0.0, lax.rsqrt(jnp.where(d > 0.0, d, 1.0)), 0.0)


_dis = pl.pallas_call(
    _dis_body,
    grid=(1,),
    in_specs=[pl.BlockSpec((16, 1280), lambda i: (0, 0))],
    out_specs=pl.BlockSpec((8, 1280), lambda i: (0, 0)),
    out_shape=jax.ShapeDtypeStruct((8, 1280), jnp.float32),
)


_BN = 512
_NB = 20
_PB = _NPAD // _BN


def _comb_body(a, b, o):
    o[...] = a[...] + b[...]


_comb = pl.pallas_call(
    _comb_body,
    grid=(_NB,),
    in_specs=[
        pl.BlockSpec((_BN, _D), lambda i: (i, 0)),
        pl.BlockSpec((_BN, _D), lambda i: (i + _PB, 0)),
    ],
    out_specs=pl.BlockSpec((_BN, _D), lambda i: (i, 0)),
    out_shape=jax.ShapeDtypeStruct((_N, _D), jnp.float32),
)


def _dense1_body(x, u1, u2a, u2b, h, v1, v2a, v2b, wz, wr, bz, br, z_o, hr_o):
    xx = x[...]
    hh = h[...]
    cat = jnp.concatenate(
        [xx, u1[...], u2a[...] + u2b[...], hh, v1[...], v2a[...] + v2b[...]],
        axis=1)
    z = jax.nn.sigmoid(
        jnp.dot(cat, wz[...], preferred_element_type=jnp.float32) + bz[...])
    r = jax.nn.sigmoid(
        jnp.dot(cat, wr[...], preferred_element_type=jnp.float32) + br[...])
    z_o[...] = z
    hr_o[...] = hh * r


_dense1 = pl.pallas_call(
    _dense1_body,
    grid=(_NB,),
    in_specs=[
        pl.BlockSpec((_BN, _D), lambda i: (i, 0)),
        pl.BlockSpec((_BN, _D), lambda i: (i, 0)),
        pl.BlockSpec((_BN, _D), lambda i: (i, 0)),
        pl.BlockSpec((_BN, _D), lambda i: (i + _PB, 0)),
        pl.BlockSpec((_BN, _D), lambda i: (i, 0)),
        pl.BlockSpec((_BN, _D), lambda i: (i, 0)),
        pl.BlockSpec((_BN, _D), lambda i: (i, 0)),
        pl.BlockSpec((_BN, _D), lambda i: (i + _PB, 0)),
        pl.BlockSpec((6 * _D, _D), lambda i: (0, 0)),
        pl.BlockSpec((6 * _D, _D), lambda i: (0, 0)),
        pl.BlockSpec((1, _D), lambda i: (0, 0)),
        pl.BlockSpec((1, _D), lambda i: (0, 0)),
    ],
    out_specs=[
        pl.BlockSpec((_BN, _D), lambda i: (i, 0)),
        pl.BlockSpec((_BN, _D), lambda i: (i, 0)),
    ],
    out_shape=[
        jax.ShapeDtypeStruct((_N, _D), jnp.float32),
        jax.ShapeDtypeStruct((_N, _D), jnp.float32),
    ],
)


def _dense2_body(x, u1, u2a, u2b, hr, p1, p2a, p2b, z, h, wh, bh, o):
    cat = jnp.concatenate(
        [x[...], u1[...], u2a[...] + u2b[...],
         hr[...], p1[...], p2a[...] + p2b[...]], axis=1)
    ht = jnp.tanh(
        jnp.dot(cat, wh[...], preferred_element_type=jnp.float32) + bh[...])
    zz = z[...]
    o[...] = zz * h[...] + (1.0 - zz) * ht


_dense2 = pl.pallas_call(
    _dense2_body,
    grid=(_NB,),
    in_specs=[
        pl.BlockSpec((_BN, _D), lambda i: (i, 0)),
        pl.BlockSpec((_BN, _D), lambda i: (i, 0)),
        pl.BlockSpec((_BN, _D), lambda i: (i, 0)),
        pl.BlockSpec((_BN, _D), lambda i: (i + _PB, 0)),
        pl.BlockSpec((_BN, _D), lambda i: (i, 0)),
        pl.BlockSpec((_BN, _D), lambda i: (i, 0)),
        pl.BlockSpec((_BN, _D), lambda i: (i, 0)),
        pl.BlockSpec((_BN, _D), lambda i: (i + _PB, 0)),
        pl.BlockSpec((_BN, _D), lambda i: (i, 0)),
        pl.BlockSpec((_BN, _D), lambda i: (i, 0)),
        pl.BlockSpec((6 * _D, _D), lambda i: (0, 0)),
        pl.BlockSpec((1, _D), lambda i: (0, 0)),
    ],
    out_specs=pl.BlockSpec((_BN, _D), lambda i: (i, 0)),
    out_shape=jax.ShapeDtypeStruct((_N, _D), jnp.float32),
)


def _gate_weights(Wx, Wh):
    return jnp.concatenate([Wx[0] - Wx[2], Wx[1], 2.0 * Wx[2],
                            Wh[0] - Wh[2], Wh[1], 2.0 * Wh[2]], axis=0)


def kernel(X, edge_index, edge_weight, H,
           Wxz, bxz, Whz, bhz, Wxr, bxr, Whr, bhr, Wxh, bxh, Whh, bhh):
    pad = _EPAD - _E
    zi = jnp.zeros((pad,), jnp.int32)
    src2 = jnp.concatenate([edge_index[0], zi]).reshape(_ER, _C)
    dst2 = jnp.concatenate([edge_index[1], zi]).reshape(_ER, _C)
    w2 = jnp.concatenate([edge_weight, jnp.zeros((pad,), jnp.float32)]).reshape(_ER, _C)

    degp = _deg(src2, w2)
    dis = _dis(degp.reshape(16, 1280)).reshape(_NPAD)
    nn2 = _norm(dis, src2, dst2, w2)

    u1p = _lap(X, src2, dst2, nn2)
    u1 = _comb(u1p, u1p)
    u2p = _lap(u1, src2, dst2, nn2)
    v1p = _lap(H, src2, dst2, nn2)
    v1 = _comb(v1p, v1p)
    v2p = _lap(v1, src2, dst2, nn2)

    Wz = _gate_weights(Wxz, Whz)
    Wr = _gate_weights(Wxr, Whr)
    Wh = _gate_weights(Wxh, Whh)
    bz = (bxz + bhz).reshape(1, _D)
    br = (bxr + bhr).reshape(1, _D)
    bh = (bxh + bhh).reshape(1, _D)

    Z, HR = _dense1(X, u1, u2p, u2p, H, v1, v2p, v2p, Wz, Wr, bz, br)

    p1p = _lap(HR, src2, dst2, nn2)
    p1 = _comb(p1p, p1p)
    p2p = _lap(p1, src2, dst2, nn2)

    return _dense2(X, u1, u2p, u2p, HR, p1, p2p, p2p, Z, H, Wh, bh)


# speedup vs baseline: 1.1758x; 1.0431x over previous
import functools

import jax
import jax.numpy as jnp
from jax import lax
from jax.experimental import pallas as pl
from jax.experimental.pallas import tpu as pltpu
from jax.experimental.pallas import tpu_sc as plsc

_N = 10000
_D = 128
_E = 320000
_C = 80
_EPAD = 327680
_ER = _EPAD // _C
_RPT = _ER // 32
_LSTG = 16
_NPAD = 10240

_mesh = plsc.VectorSubcoreMesh(core_axis_name="c", subcore_axis_name="s")

_Z16 = functools.partial(jnp.zeros, (16,), jnp.float32)


@functools.partial(
    pl.kernel,
    out_type=jax.ShapeDtypeStruct((2 * _NPAD,), jnp.float32),
    mesh=_mesh,
    scratch_types=[
        pltpu.VMEM((_RPT, _C), jnp.int32),
        pltpu.VMEM((_RPT, _C), jnp.float32),
        pltpu.VMEM((640,), jnp.float32),
        pltpu.VMEM_SHARED((_NPAD,), jnp.float32),
    ],
)
def _deg(src_hbm, w_hbm, out_hbm, sidx, w_all, zbuf, acc):
    cid = lax.axis_index("c")
    sid = lax.axis_index("s")
    wid = cid * 16 + sid
    for j in range(40):
        zbuf[pl.ds(j * 16, 16)] = _Z16()
    off = pl.multiple_of(sid * 640, 8)
    pltpu.sync_copy(zbuf, acc.at[pl.ds(off, 640)])
    plsc.subcore_barrier()
    rbase = pl.multiple_of(wid * _RPT, 1)
    pltpu.sync_copy(src_hbm.at[pl.ds(rbase, _RPT)], sidx)
    pltpu.sync_copy(w_hbm.at[pl.ds(rbase, _RPT)], w_all)

    def chunk(k, carry):
        pltpu.sync_copy(w_all.at[k], acc.at[sidx.at[k]], add=True)
        return carry

    lax.fori_loop(0, _RPT, chunk, 0)
    plsc.subcore_barrier()
    obase = pl.multiple_of(cid * _NPAD + sid * 640, 8)
    pltpu.sync_copy(acc.at[pl.ds(off, 640)], out_hbm.at[pl.ds(obase, 640)])


@functools.partial(
    pl.kernel,
    out_type=jax.ShapeDtypeStruct((_ER, _C), jnp.float32),
    mesh=_mesh,
    scratch_types=[
        pltpu.VMEM((_RPT, _C), jnp.int32),
        pltpu.VMEM((_RPT, _C), jnp.int32),
        pltpu.VMEM((_RPT, _C), jnp.float32),
        pltpu.VMEM((_RPT, _C), jnp.float32),
        pltpu.VMEM((_C,), jnp.float32),
        pltpu.VMEM((_C,), jnp.float32),
        pltpu.VMEM_SHARED((_NPAD,), jnp.float32),
    ],
)
def _norm(dis_hbm, src_hbm, dst_hbm, w_hbm, out_hbm,
          sidx, didx, w_all, nnbuf, sv, dg, sdis):
    cid = lax.axis_index("c")
    sid = lax.axis_index("s")
    wid = cid * 16 + sid
    off = pl.multiple_of(sid * 640, 8)
    pltpu.sync_copy(dis_hbm.at[pl.ds(off, 640)], sdis.at[pl.ds(off, 640)])
    plsc.subcore_barrier()
    rbase = pl.multiple_of(wid * _RPT, 1)
    pltpu.sync_copy(src_hbm.at[pl.ds(rbase, _RPT)], sidx)
    pltpu.sync_copy(dst_hbm.at[pl.ds(rbase, _RPT)], didx)
    pltpu.sync_copy(w_hbm.at[pl.ds(rbase, _RPT)], w_all)

    def chunk(k, carry):
        pltpu.sync_copy(sdis.at[sidx.at[k]], sv)
        pltpu.sync_copy(sdis.at[didx.at[k]], dg)
        for j in range(_C // 16):
            s = sv[pl.ds(j * 16, 16)]
            dd = dg[pl.ds(j * 16, 16)]
            ww = w_all[k, pl.ds(j * 16, 16)]
            nnbuf[k, pl.ds(j * 16, 16)] = -(s * ww * dd)
        return carry

    lax.fori_loop(0, _RPT, chunk, 0)
    pltpu.sync_copy(nnbuf, out_hbm.at[pl.ds(rbase, _RPT)])


@functools.partial(
    pl.kernel,
    out_type=jax.ShapeDtypeStruct((2 * _NPAD, _D), jnp.float32),
    mesh=_mesh,
    scratch_types=[
        pltpu.VMEM((_LSTG, _C), jnp.int32),
        pltpu.VMEM((_LSTG, _C), jnp.int32),
        pltpu.VMEM((_LSTG, _C), jnp.float32),
        pltpu.VMEM((_C, _D), jnp.float32),
        pltpu.VMEM((_C, _D), jnp.float32),
        pltpu.SemaphoreType.DMA,
        pltpu.SemaphoreType.DMA,
        pltpu.VMEM_SHARED((_NPAD, _D), jnp.float32),
    ],
)
def _lap(v_hbm, src_hbm, dst_hbm, nn_hbm, out_hbm,
         sidx, didx, nn_all, rows0, rows1, sem0, sem1, acc):
    cid = lax.axis_index("c")
    sid = lax.axis_index("s")
    wid = cid * 16 + sid

    def zrow(i, carry):
        for j in range(_D // 16):
            rows0[i, pl.ds(j * 16, 16)] = _Z16()
        return carry

    lax.fori_loop(0, _C, zrow, 0)
    for t in range(640 // _C):
        pltpu.sync_copy(
            rows0, acc.at[pl.ds(pl.multiple_of(sid * 640 + t * _C, 8), _C)])
    plsc.subcore_barrier()
    rbase = pl.multiple_of(wid * _RPT, 8)

    def scale(rows, ks):
        for g in range(_C // 16):
            nv = nn_all[ks, pl.ds(g * 16, 16)]
            for lane in range(16):
                ri = g * 16 + lane
                sc = nv[lane]
                for j in range(_D // 16):
                    rows[ri, pl.ds(j * 16, 16)] = rows[ri, pl.ds(j * 16, 16)] * sc

    def pair(kk, carry):
        ks0 = (2 * kk) % _LSTG
        ks1 = ks0 + 1

        @pl.when(ks0 == 0)
        def _stage():
            rb = pl.multiple_of(rbase + 2 * kk, 8)
            pltpu.sync_copy(src_hbm.at[pl.ds(rb, _LSTG)], sidx)
            pltpu.sync_copy(dst_hbm.at[pl.ds(rb, _LSTG)], didx)
            pltpu.sync_copy(nn_hbm.at[pl.ds(rb, _LSTG)], nn_all)
            pltpu.async_copy(v_hbm.at[sidx.at[ks0]], rows0, sem0)
            pltpu.async_copy(v_hbm.at[sidx.at[ks1]], rows1, sem1)

        pltpu.make_async_copy(v_hbm.at[sidx.at[ks0]], rows0, sem0).wait()
        scale(rows0, ks0)
        pltpu.sync_copy(rows0, acc.at[didx.at[ks0]], add=True)

        @pl.when(ks0 + 2 < _LSTG)
        def _pref0():
            pltpu.async_copy(v_hbm.at[sidx.at[ks0 + 2]], rows0, sem0)

        pltpu.make_async_copy(v_hbm.at[sidx.at[ks1]], rows1, sem1).wait()
        scale(rows1, ks1)
        pltpu.sync_copy(rows1, acc.at[didx.at[ks1]], add=True)

        @pl.when(ks0 + 3 < _LSTG)
        def _pref1():
            pltpu.async_copy(v_hbm.at[sidx.at[ks0 + 3]], rows1, sem1)

        return carry

    lax.fori_loop(0, _RPT // 2, pair, 0)
    plsc.subcore_barrier()
    pltpu.sync_copy(acc.at[pl.ds(pl.multiple_of(sid * 640, 8), 640)],
                    out_hbm.at[pl.ds(pl.multiple_of(cid * _NPAD + sid * 640, 8), 640)])


def _dis_body(dp, o):
    d = dp[0:8, :] + dp[8:16, :]
    o[...] = jnp.where(d > 0.0, lax.rsqrt(jnp.where(d > 0.0, d, 1.0)), 0.0)


_dis = pl.pallas_call(
    _dis_body,
    grid=(1,),
    in_specs=[pl.BlockSpec((16, 1280), lambda i: (0, 0))],
    out_specs=pl.BlockSpec((8, 1280), lambda i: (0, 0)),
    out_shape=jax.ShapeDtypeStruct((8, 1280), jnp.float32),
)


_BN = 512
_NB = 20
_PB = _NPAD // _BN


def _comb_body(a, b, o):
    o[...] = a[...] + b[...]


_comb = pl.pallas_call(
    _comb_body,
    grid=(_NB,),
    in_specs=[
        pl.BlockSpec((_BN, _D), lambda i: (i, 0)),
        pl.BlockSpec((_BN, _D), lambda i: (i + _PB, 0)),
    ],
    out_specs=pl.BlockSpec((_BN, _D), lambda i: (i, 0)),
    out_shape=jax.ShapeDtypeStruct((_N, _D), jnp.float32),
)


def _dense1_body(x, u1, u2a, u2b, h, v1, v2a, v2b, wz, wr, bz, br, z_o, hr_o):
    xx = x[...]
    hh = h[...]
    cat = jnp.concatenate(
        [xx, u1[...], u2a[...] + u2b[...], hh, v1[...], v2a[...] + v2b[...]],
        axis=1)
    z = jax.nn.sigmoid(
        jnp.dot(cat, wz[...], preferred_element_type=jnp.float32) + bz[...])
    r = jax.nn.sigmoid(
        jnp.dot(cat, wr[...], preferred_element_type=jnp.float32) + br[...])
    z_o[...] = z
    hr_o[...] = hh * r


_dense1 = pl.pallas_call(
    _dense1_body,
    grid=(_NB,),
    in_specs=[
        pl.BlockSpec((_BN, _D), lambda i: (i, 0)),
        pl.BlockSpec((_BN, _D), lambda i: (i, 0)),
        pl.BlockSpec((_BN, _D), lambda i: (i, 0)),
        pl.BlockSpec((_BN, _D), lambda i: (i + _PB, 0)),
        pl.BlockSpec((_BN, _D), lambda i: (i, 0)),
        pl.BlockSpec((_BN, _D), lambda i: (i, 0)),
        pl.BlockSpec((_BN, _D), lambda i: (i, 0)),
        pl.BlockSpec((_BN, _D), lambda i: (i + _PB, 0)),
        pl.BlockSpec((6 * _D, _D), lambda i: (0, 0)),
        pl.BlockSpec((6 * _D, _D), lambda i: (0, 0)),
        pl.BlockSpec((1, _D), lambda i: (0, 0)),
        pl.BlockSpec((1, _D), lambda i: (0, 0)),
    ],
    out_specs=[
        pl.BlockSpec((_BN, _D), lambda i: (i, 0)),
        pl.BlockSpec((_BN, _D), lambda i: (i, 0)),
    ],
    out_shape=[
        jax.ShapeDtypeStruct((_N, _D), jnp.float32),
        jax.ShapeDtypeStruct((_N, _D), jnp.float32),
    ],
)


def _dense2_body(x, u1, u2a, u2b, hr, p1, p2a, p2b, z, h, wh, bh, o):
    cat = jnp.concatenate(
        [x[...], u1[...], u2a[...] + u2b[...],
         hr[...], p1[...], p2a[...] + p2b[...]], axis=1)
    ht = jnp.tanh(
        jnp.dot(cat, wh[...], preferred_element_type=jnp.float32) + bh[...])
    zz = z[...]
    o[...] = zz * h[...] + (1.0 - zz) * ht


_dense2 = pl.pallas_call(
    _dense2_body,
    grid=(_NB,),
    in_specs=[
        pl.BlockSpec((_BN, _D), lambda i: (i, 0)),
        pl.BlockSpec((_BN, _D), lambda i: (i, 0)),
        pl.BlockSpec((_BN, _D), lambda i: (i, 0)),
        pl.BlockSpec((_BN, _D), lambda i: (i + _PB, 0)),
        pl.BlockSpec((_BN, _D), lambda i: (i, 0)),
        pl.BlockSpec((_BN, _D), lambda i: (i, 0)),
        pl.BlockSpec((_BN, _D), lambda i: (i, 0)),
        pl.BlockSpec((_BN, _D), lambda i: (i + _PB, 0)),
        pl.BlockSpec((_BN, _D), lambda i: (i, 0)),
        pl.BlockSpec((_BN, _D), lambda i: (i, 0)),
        pl.BlockSpec((6 * _D, _D), lambda i: (0, 0)),
        pl.BlockSpec((1, _D), lambda i: (0, 0)),
    ],
    out_specs=pl.BlockSpec((_BN, _D), lambda i: (i, 0)),
    out_shape=jax.ShapeDtypeStruct((_N, _D), jnp.float32),
)


def _gate_weights(Wx, Wh):
    return jnp.concatenate([Wx[0] - Wx[2], Wx[1], 2.0 * Wx[2],
                            Wh[0] - Wh[2], Wh[1], 2.0 * Wh[2]], axis=0)


def kernel(X, edge_index, edge_weight, H,
           Wxz, bxz, Whz, bhz, Wxr, bxr, Whr, bhr, Wxh, bxh, Whh, bhh):
    pad = _EPAD - _E
    zi = jnp.zeros((pad,), jnp.int32)
    src2 = jnp.concatenate([edge_index[0], zi]).reshape(_ER, _C)
    dst2 = jnp.concatenate([edge_index[1], zi]).reshape(_ER, _C)
    w2 = jnp.concatenate([edge_weight, jnp.zeros((pad,), jnp.float32)]).reshape(_ER, _C)

    degp = _deg(src2, w2)
    dis = _dis(degp.reshape(16, 1280)).reshape(_NPAD)
    nn2 = _norm(dis, src2, dst2, w2)

    u1p = _lap(X, src2, dst2, nn2)
    u1 = _comb(u1p, u1p)
    u2p = _lap(u1, src2, dst2, nn2)
    v1p = _lap(H, src2, dst2, nn2)
    v1 = _comb(v1p, v1p)
    v2p = _lap(v1, src2, dst2, nn2)

    Wz = _gate_weights(Wxz, Whz)
    Wr = _gate_weights(Wxr, Whr)
    Wh = _gate_weights(Wxh, Whh)
    bz = (bxz + bhz).reshape(1, _D)
    br = (bxr + bhr).reshape(1, _D)
    bh = (bxh + bhh).reshape(1, _D)

    Z, HR = _dense1(X, u1, u2p, u2p, H, v1, v2p, v2p, Wz, Wr, bz, br)

    p1p = _lap(HR, src2, dst2, nn2)
    p1 = _comb(p1p, p1p)
    p2p = _lap(p1, src2, dst2, nn2)

    return _dense2(X, u1, u2p, u2p, HR, p1, p2p, p2p, Z, H, Wh, bh)


# speedup vs baseline: 1.2106x; 1.0295x over previous
import functools

import jax
import jax.numpy as jnp
from jax import lax
from jax.experimental import pallas as pl
from jax.experimental.pallas import tpu as pltpu
from jax.experimental.pallas import tpu_sc as plsc

_N = 10000
_D = 128
_E = 320000
_C = 80
_EPAD = 327680
_ER = _EPAD // _C
_RPT = _ER // 32
_LSTG = 32
_NPAD = 10240

_mesh = plsc.VectorSubcoreMesh(core_axis_name="c", subcore_axis_name="s")

_Z16 = functools.partial(jnp.zeros, (16,), jnp.float32)


@functools.partial(
    pl.kernel,
    out_type=jax.ShapeDtypeStruct((2 * _NPAD,), jnp.float32),
    mesh=_mesh,
    scratch_types=[
        pltpu.VMEM((_RPT, _C), jnp.int32),
        pltpu.VMEM((_RPT, _C), jnp.float32),
        pltpu.VMEM((640,), jnp.float32),
        pltpu.VMEM_SHARED((_NPAD,), jnp.float32),
    ],
)
def _deg(src_hbm, w_hbm, out_hbm, sidx, w_all, zbuf, acc):
    cid = lax.axis_index("c")
    sid = lax.axis_index("s")
    wid = cid * 16 + sid
    for j in range(40):
        zbuf[pl.ds(j * 16, 16)] = _Z16()
    off = pl.multiple_of(sid * 640, 8)
    pltpu.sync_copy(zbuf, acc.at[pl.ds(off, 640)])
    plsc.subcore_barrier()
    rbase = pl.multiple_of(wid * _RPT, 1)
    pltpu.sync_copy(src_hbm.at[pl.ds(rbase, _RPT)], sidx)
    pltpu.sync_copy(w_hbm.at[pl.ds(rbase, _RPT)], w_all)

    def chunk(k, carry):
        pltpu.sync_copy(w_all.at[k], acc.at[sidx.at[k]], add=True)
        return carry

    lax.fori_loop(0, _RPT, chunk, 0)
    plsc.subcore_barrier()
    obase = pl.multiple_of(cid * _NPAD + sid * 640, 8)
    pltpu.sync_copy(acc.at[pl.ds(off, 640)], out_hbm.at[pl.ds(obase, 640)])


@functools.partial(
    pl.kernel,
    out_type=jax.ShapeDtypeStruct((_ER, _C), jnp.float32),
    mesh=_mesh,
    scratch_types=[
        pltpu.VMEM((_RPT, _C), jnp.int32),
        pltpu.VMEM((_RPT, _C), jnp.int32),
        pltpu.VMEM((_RPT, _C), jnp.float32),
        pltpu.VMEM((_RPT, _C), jnp.float32),
        pltpu.VMEM((_C,), jnp.float32),
        pltpu.VMEM((_C,), jnp.float32),
        pltpu.VMEM_SHARED((_NPAD,), jnp.float32),
    ],
)
def _norm(dis_hbm, src_hbm, dst_hbm, w_hbm, out_hbm,
          sidx, didx, w_all, nnbuf, sv, dg, sdis):
    cid = lax.axis_index("c")
    sid = lax.axis_index("s")
    wid = cid * 16 + sid
    off = pl.multiple_of(sid * 640, 8)
    pltpu.sync_copy(dis_hbm.at[pl.ds(off, 640)], sdis.at[pl.ds(off, 640)])
    plsc.subcore_barrier()
    rbase = pl.multiple_of(wid * _RPT, 1)
    pltpu.sync_copy(src_hbm.at[pl.ds(rbase, _RPT)], sidx)
    pltpu.sync_copy(dst_hbm.at[pl.ds(rbase, _RPT)], didx)
    pltpu.sync_copy(w_hbm.at[pl.ds(rbase, _RPT)], w_all)

    def chunk(k, carry):
        pltpu.sync_copy(sdis.at[sidx.at[k]], sv)
        pltpu.sync_copy(sdis.at[didx.at[k]], dg)
        for j in range(_C // 16):
            s = sv[pl.ds(j * 16, 16)]
            dd = dg[pl.ds(j * 16, 16)]
            ww = w_all[k, pl.ds(j * 16, 16)]
            nnbuf[k, pl.ds(j * 16, 16)] = -(s * ww * dd)
        return carry

    lax.fori_loop(0, _RPT, chunk, 0)
    pltpu.sync_copy(nnbuf, out_hbm.at[pl.ds(rbase, _RPT)])


@functools.partial(
    pl.kernel,
    out_type=jax.ShapeDtypeStruct((2 * _NPAD, _D), jnp.float32),
    mesh=_mesh,
    scratch_types=[
        pltpu.VMEM((_LSTG, _C), jnp.int32),
        pltpu.VMEM((_LSTG, _C), jnp.int32),
        pltpu.VMEM((_LSTG, _C), jnp.float32),
        pltpu.VMEM((_C, _D), jnp.float32),
        pltpu.VMEM((_C, _D), jnp.float32),
        pltpu.SemaphoreType.DMA,
        pltpu.SemaphoreType.DMA,
        pltpu.VMEM_SHARED((_NPAD, _D), jnp.float32),
    ],
)
def _lap(v_hbm, src_hbm, dst_hbm, nn_hbm, out_hbm,
         sidx, didx, nn_all, rows0, rows1, sem0, sem1, acc):
    cid = lax.axis_index("c")
    sid = lax.axis_index("s")
    wid = cid * 16 + sid

    def zrow(i, carry):
        for j in range(_D // 16):
            rows0[i, pl.ds(j * 16, 16)] = _Z16()
        return carry

    lax.fori_loop(0, _C, zrow, 0)
    for t in range(640 // _C):
        pltpu.sync_copy(
            rows0, acc.at[pl.ds(pl.multiple_of(sid * 640 + t * _C, 8), _C)])
    plsc.subcore_barrier()
    rbase = pl.multiple_of(wid * _RPT, 8)

    def scale(rows, ks):
        for g in range(_C // 16):
            nv = nn_all[ks, pl.ds(g * 16, 16)]
            for lane in range(16):
                ri = g * 16 + lane
                sc = nv[lane]
                for j in range(_D // 16):
                    rows[ri, pl.ds(j * 16, 16)] = rows[ri, pl.ds(j * 16, 16)] * sc

    def pair(kk, carry):
        ks0 = (2 * kk) % _LSTG
        ks1 = ks0 + 1

        @pl.when(ks0 == 0)
        def _stage():
            rb = pl.multiple_of(rbase + 2 * kk, 8)
            pltpu.sync_copy(src_hbm.at[pl.ds(rb, _LSTG)], sidx)
            pltpu.sync_copy(dst_hbm.at[pl.ds(rb, _LSTG)], didx)
            pltpu.sync_copy(nn_hbm.at[pl.ds(rb, _LSTG)], nn_all)
            pltpu.async_copy(v_hbm.at[sidx.at[ks0]], rows0, sem0)
            pltpu.async_copy(v_hbm.at[sidx.at[ks1]], rows1, sem1)

        pltpu.make_async_copy(v_hbm.at[sidx.at[ks0]], rows0, sem0).wait()
        scale(rows0, ks0)
        pltpu.sync_copy(rows0, acc.at[didx.at[ks0]], add=True)

        @pl.when(ks0 + 2 < _LSTG)
        def _pref0():
            pltpu.async_copy(v_hbm.at[sidx.at[ks0 + 2]], rows0, sem0)

        pltpu.make_async_copy(v_hbm.at[sidx.at[ks1]], rows1, sem1).wait()
        scale(rows1, ks1)
        pltpu.sync_copy(rows1, acc.at[didx.at[ks1]], add=True)

        @pl.when(ks0 + 3 < _LSTG)
        def _pref1():
            pltpu.async_copy(v_hbm.at[sidx.at[ks0 + 3]], rows1, sem1)

        return carry

    lax.fori_loop(0, _RPT // 2, pair, 0)
    plsc.subcore_barrier()
    pltpu.sync_copy(acc.at[pl.ds(pl.multiple_of(sid * 640, 8), 640)],
                    out_hbm.at[pl.ds(pl.multiple_of(cid * _NPAD + sid * 640, 8), 640)])


def _dis_body(dp, o):
    d = dp[0:8, :] + dp[8:16, :]
    o[...] = jnp.where(d > 0.0, lax.rsqrt(jnp.where(d > 0.0, d, 1.0)), 0.0)


_dis = pl.pallas_call(
    _dis_body,
    grid=(1,),
    in_specs=[pl.BlockSpec((16, 1280), lambda i: (0, 0))],
    out_specs=pl.BlockSpec((8, 1280), lambda i: (0, 0)),
    out_shape=jax.ShapeDtypeStruct((8, 1280), jnp.float32),
)


_BN = 512
_NB = 20
_PB = _NPAD // _BN


def _comb_body(a, b, o):
    o[...] = a[...] + b[...]


_comb = pl.pallas_call(
    _comb_body,
    grid=(_NB,),
    in_specs=[
        pl.BlockSpec((_BN, _D), lambda i: (i, 0)),
        pl.BlockSpec((_BN, _D), lambda i: (i + _PB, 0)),
    ],
    out_specs=pl.BlockSpec((_BN, _D), lambda i: (i, 0)),
    out_shape=jax.ShapeDtypeStruct((_N, _D), jnp.float32),
)


def _dense1_body(x, u1, u2a, u2b, h, v1, v2a, v2b, wz, wr, bz, br, z_o, hr_o):
    xx = x[...]
    hh = h[...]
    cat = jnp.concatenate(
        [xx, u1[...], u2a[...] + u2b[...], hh, v1[...], v2a[...] + v2b[...]],
        axis=1)
    z = jax.nn.sigmoid(
        jnp.dot(cat, wz[...], preferred_element_type=jnp.float32) + bz[...])
    r = jax.nn.sigmoid(
        jnp.dot(cat, wr[...], preferred_element_type=jnp.float32) + br[...])
    z_o[...] = z
    hr_o[...] = hh * r


_dense1 = pl.pallas_call(
    _dense1_body,
    grid=(_NB,),
    in_specs=[
        pl.BlockSpec((_BN, _D), lambda i: (i, 0)),
        pl.BlockSpec((_BN, _D), lambda i: (i, 0)),
        pl.BlockSpec((_BN, _D), lambda i: (i, 0)),
        pl.BlockSpec((_BN, _D), lambda i: (i + _PB, 0)),
        pl.BlockSpec((_BN, _D), lambda i: (i, 0)),
        pl.BlockSpec((_BN, _D), lambda i: (i, 0)),
        pl.BlockSpec((_BN, _D), lambda i: (i, 0)),
        pl.BlockSpec((_BN, _D), lambda i: (i + _PB, 0)),
        pl.BlockSpec((6 * _D, _D), lambda i: (0, 0)),
        pl.BlockSpec((6 * _D, _D), lambda i: (0, 0)),
        pl.BlockSpec((1, _D), lambda i: (0, 0)),
        pl.BlockSpec((1, _D), lambda i: (0, 0)),
    ],
    out_specs=[
        pl.BlockSpec((_BN, _D), lambda i: (i, 0)),
        pl.BlockSpec((_BN, _D), lambda i: (i, 0)),
    ],
    out_shape=[
        jax.ShapeDtypeStruct((_N, _D), jnp.float32),
        jax.ShapeDtypeStruct((_N, _D), jnp.float32),
    ],
)


def _dense2_body(x, u1, u2a, u2b, hr, p1, p2a, p2b, z, h, wh, bh, o):
    cat = jnp.concatenate(
        [x[...], u1[...], u2a[...] + u2b[...],
         hr[...], p1[...], p2a[...] + p2b[...]], axis=1)
    ht = jnp.tanh(
        jnp.dot(cat, wh[...], preferred_element_type=jnp.float32) + bh[...])
    zz = z[...]
    o[...] = zz * h[...] + (1.0 - zz) * ht


_dense2 = pl.pallas_call(
    _dense2_body,
    grid=(_NB,),
    in_specs=[
        pl.BlockSpec((_BN, _D), lambda i: (i, 0)),
        pl.BlockSpec((_BN, _D), lambda i: (i, 0)),
        pl.BlockSpec((_BN, _D), lambda i: (i, 0)),
        pl.BlockSpec((_BN, _D), lambda i: (i + _PB, 0)),
        pl.BlockSpec((_BN, _D), lambda i: (i, 0)),
        pl.BlockSpec((_BN, _D), lambda i: (i, 0)),
        pl.BlockSpec((_BN, _D), lambda i: (i, 0)),
        pl.BlockSpec((_BN, _D), lambda i: (i + _PB, 0)),
        pl.BlockSpec((_BN, _D), lambda i: (i, 0)),
        pl.BlockSpec((_BN, _D), lambda i: (i, 0)),
        pl.BlockSpec((6 * _D, _D), lambda i: (0, 0)),
        pl.BlockSpec((1, _D), lambda i: (0, 0)),
    ],
    out_specs=pl.BlockSpec((_BN, _D), lambda i: (i, 0)),
    out_shape=jax.ShapeDtypeStruct((_N, _D), jnp.float32),
)


def _gate_weights(Wx, Wh):
    return jnp.concatenate([Wx[0] - Wx[2], Wx[1], 2.0 * Wx[2],
                            Wh[0] - Wh[2], Wh[1], 2.0 * Wh[2]], axis=0)


def kernel(X, edge_index, edge_weight, H,
           Wxz, bxz, Whz, bhz, Wxr, bxr, Whr, bhr, Wxh, bxh, Whh, bhh):
    pad = _EPAD - _E
    zi = jnp.zeros((pad,), jnp.int32)
    src2 = jnp.concatenate([edge_index[0], zi]).reshape(_ER, _C)
    dst2 = jnp.concatenate([edge_index[1], zi]).reshape(_ER, _C)
    w2 = jnp.concatenate([edge_weight, jnp.zeros((pad,), jnp.float32)]).reshape(_ER, _C)

    degp = _deg(src2, w2)
    dis = _dis(degp.reshape(16, 1280)).reshape(_NPAD)
    nn2 = _norm(dis, src2, dst2, w2)

    u1p = _lap(X, src2, dst2, nn2)
    u1 = _comb(u1p, u1p)
    u2p = _lap(u1, src2, dst2, nn2)
    v1p = _lap(H, src2, dst2, nn2)
    v1 = _comb(v1p, v1p)
    v2p = _lap(v1, src2, dst2, nn2)

    Wz = _gate_weights(Wxz, Whz)
    Wr = _gate_weights(Wxr, Whr)
    Wh = _gate_weights(Wxh, Whh)
    bz = (bxz + bhz).reshape(1, _D)
    br = (bxr + bhr).reshape(1, _D)
    bh = (bxh + bhh).reshape(1, _D)

    Z, HR = _dense1(X, u1, u2p, u2p, H, v1, v2p, v2p, Wz, Wr, bz, br)

    p1p = _lap(HR, src2, dst2, nn2)
    p1 = _comb(p1p, p1p)
    p2p = _lap(p1, src2, dst2, nn2)

    return _dense2(X, u1, u2p, u2p, HR, p1, p2p, p2p, Z, H, Wh, bh)
